# baseline (speedup 1.0000x reference)
import functools

import jax
import jax.numpy as jnp
from jax import lax
from jax.experimental import pallas as pl
from jax.experimental.pallas import tpu as pltpu
from jax.experimental.pallas import tpu_sc as plsc

N = 10000
E = 320000
D = 128

NC = 2
NS = 16
NW = NC * NS

NPAD = 10240
RPT = NPAD // NS
EPAD = 327680

KD = 128
NCHD = EPAD // NW // KD

KA = 80
EPTA = EPAD // NW
NCHA = EPTA // KA
NPAIR = NCHA // 2

BN = 10000
GRID = N // BN


def _mesh():
    return plsc.VectorSubcoreMesh(
        core_axis_name="c", subcore_axis_name="s",
        num_cores=NC, num_subcores=NS)


def _unpack_chunk(pk, j, sbuf, dbuf, k, soff):
    for g in range(k // 16):
        pv = pk[j, pl.ds(g * 16, 16)]
        sbuf[pl.ds(g * 16, 16)] = (pv & 0xFFFF) + soff
        dbuf[pl.ds(g * 16, 16)] = pv >> 16


def _sc_degree(pkd, zrow):
    @functools.partial(
        pl.kernel,
        out_type=jax.ShapeDtypeStruct((NC, NPAD), jnp.float32),
        mesh=_mesh(),
        scratch_types=[
            pltpu.VMEM((NCHD, KD), jnp.int32),
            pltpu.VMEM((KD,), jnp.int32),
            pltpu.VMEM((KD,), jnp.int32),
            pltpu.VMEM((KD,), jnp.float32),
            pltpu.VMEM_SHARED((NPAD,), jnp.float32),
        ],
    )
    def deg_kernel(pk_h, zrow_h, out_h, pk, sbuf, dbuf, ones_v, deg_sh):
        c = lax.axis_index("c")
        s = lax.axis_index("s")
        w = c * NS + s
        pltpu.sync_copy(zrow_h, deg_sh.at[pl.ds(s * RPT, RPT)])
        pltpu.sync_copy(pk_h.at[w], pk)
        for i in range(KD // 16):
            ones_v[pl.ds(i * 16, 16)] = jnp.ones((16,), jnp.float32)
        plsc.subcore_barrier()

        def body(j, carry):
            _unpack_chunk(pk, j, sbuf, dbuf, KD, 0)
            pltpu.sync_copy(ones_v, deg_sh.at[dbuf], add=True)
            return carry

        lax.fori_loop(0, NCHD, body, 0)
        plsc.subcore_barrier()
        pltpu.sync_copy(deg_sh.at[pl.ds(s * RPT, RPT)],
                        out_h.at[c, pl.ds(s * RPT, RPT)])

    return deg_kernel(pkd, zrow)


def _sc_agg(y, pka, zblk):
    @functools.partial(
        pl.kernel,
        out_type=jax.ShapeDtypeStruct((NC, NPAD, D), jnp.float32),
        mesh=_mesh(),
        scratch_types=[
            pltpu.VMEM((NCHA, KA), jnp.int32),
            pltpu.VMEM((KA,), jnp.int32),
            pltpu.VMEM((KA,), jnp.int32),
            pltpu.VMEM((KA,), jnp.int32),
            pltpu.VMEM((KA,), jnp.int32),
            pltpu.VMEM((KA, D), jnp.float32),
            pltpu.VMEM((KA, D), jnp.float32),
            pltpu.VMEM_SHARED((NPAD, D), jnp.float32),
            pltpu.SemaphoreType.DMA,
            pltpu.SemaphoreType.DMA,
        ],
    )
    def agg_kernel(y_h, pk_h, zblk_h, out_h,
                   pk, sidx0, didx0, sidx1, didx1, rows0, rows1,
                   acc_sh, semg0, semg1):
        c = lax.axis_index("c")
        s = lax.axis_index("s")
        w = c * NS + s
        pltpu.sync_copy(pk_h.at[w], pk)
        _unpack_chunk(pk, 0, sidx0, didx0, KA, 0)
        pltpu.async_copy(y_h.at[sidx0], rows0, semg0)
        _unpack_chunk(pk, 1, sidx1, didx1, KA, 0)
        pltpu.async_copy(y_h.at[sidx1], rows1, semg1)
        last = NS - 1

        @pl.when((c == 0) & (s < last))
        def _():
            pltpu.sync_copy(y_h.at[pl.ds(s * RPT, RPT)],
                            acc_sh.at[pl.ds(s * RPT, RPT)])

        @pl.when((c == 0) & (s == last))
        def _():
            pltpu.sync_copy(y_h.at[pl.ds(last * RPT, N - last * RPT)],
                            acc_sh.at[pl.ds(last * RPT, N - last * RPT)])
            pltpu.sync_copy(zblk_h.at[pl.ds(0, NPAD - N)],
                            acc_sh.at[pl.ds(N, NPAD - N)])

        @pl.when(c == 1)
        def _():
            pltpu.sync_copy(zblk_h, acc_sh.at[pl.ds(s * RPT, RPT)])

        plsc.subcore_barrier()

        def body(i, carry):
            j = 2 * i
            pltpu.make_async_copy(y_h.at[sidx0], rows0, semg0).wait()
            pltpu.sync_copy(rows0, acc_sh.at[didx0], add=True)

            @pl.when(i < NPAIR - 1)
            def _():
                _unpack_chunk(pk, j + 2, sidx0, didx0, KA, 0)
                pltpu.async_copy(y_h.at[sidx0], rows0, semg0)

            pltpu.make_async_copy(y_h.at[sidx1], rows1, semg1).wait()
            pltpu.sync_copy(rows1, acc_sh.at[didx1], add=True)

            @pl.when(i < NPAIR - 1)
            def _():
                _unpack_chunk(pk, j + 3, sidx1, didx1, KA, 0)
                pltpu.async_copy(y_h.at[sidx1], rows1, semg1)

            return carry

        lax.fori_loop(0, NPAIR, body, 0)
        plsc.subcore_barrier()
        pltpu.sync_copy(acc_sh.at[pl.ds(s * RPT, RPT)],
                        out_h.at[c, pl.ds(s * RPT, RPT)])

    return agg_kernel(y, pka, zblk)


def _tc_matmul1(x, W1):
    def body(x_r, w_r, o_r):
        o_r[...] = jnp.dot(x_r[...], w_r[...],
                           preferred_element_type=jnp.float32)

    return pl.pallas_call(
        body,
        grid=(GRID,),
        in_specs=[pl.BlockSpec((BN, D), lambda i: (i, 0)),
                  pl.BlockSpec((D, D), lambda i: (0, 0))],
        out_specs=pl.BlockSpec((BN, D), lambda i: (i, 0)),
        out_shape=jax.ShapeDtypeStruct((N, D), jnp.float32),
    )(x, W1)


def _tc_scale1(xw, degt):
    def body(x_r, dg_r, y_r):
        dinv = lax.rsqrt(dg_r[:, 0:1] + dg_r[:, 1:2] + 1.0)
        y_r[...] = dinv * x_r[...]

    return pl.pallas_call(
        body,
        grid=(GRID,),
        in_specs=[pl.BlockSpec((BN, D), lambda i: (i, 0)),
                  pl.BlockSpec((BN, 2), lambda i: (i, 0))],
        out_specs=pl.BlockSpec((BN, D), lambda i: (i, 0)),
        out_shape=jax.ShapeDtypeStruct((N, D), jnp.float32),
    )(xw, degt)


def _tc_mid(agg, degt, b1, W2):
    def body(a_r, dg_r, b_r, w_r, o_r):
        dinv = lax.rsqrt(dg_r[:, 0:1] + dg_r[:, 1:2] + 1.0)
        z = dinv * (a_r[0] + a_r[1]) + b_r[...]
        o_r[...] = dinv * jnp.dot(z, w_r[...],
                                  preferred_element_type=jnp.float32)

    return pl.pallas_call(
        body,
        grid=(GRID,),
        in_specs=[pl.BlockSpec((NC, BN, D), lambda i: (0, i, 0)),
                  pl.BlockSpec((BN, 2), lambda i: (i, 0)),
                  pl.BlockSpec((1, D), lambda i: (0, 0)),
                  pl.BlockSpec((D, D), lambda i: (0, 0))],
        out_specs=pl.BlockSpec((BN, D), lambda i: (i, 0)),
        out_shape=jax.ShapeDtypeStruct((N, D), jnp.float32),
    )(agg, degt, b1, W2)


def _tc_final(agg, degt, b2, alpha):
    def body(a_r, dg_r, b_r, al_r, o_r):
        dinv = lax.rsqrt(dg_r[:, 0:1] + dg_r[:, 1:2] + 1.0)
        z = dinv * (a_r[0] + a_r[1]) + b_r[...]
        o_r[...] = jnp.where(z >= 0, z, al_r[...] * z)

    return pl.pallas_call(
        body,
        grid=(GRID,),
        in_specs=[pl.BlockSpec((NC, BN, D), lambda i: (0, i, 0)),
                  pl.BlockSpec((BN, 2), lambda i: (i, 0)),
                  pl.BlockSpec((1, D), lambda i: (0, 0)),
                  pl.BlockSpec((1, D), lambda i: (0, 0))],
        out_specs=pl.BlockSpec((BN, D), lambda i: (i, 0)),
        out_shape=jax.ShapeDtypeStruct((N, D), jnp.float32),
    )(agg, degt, b2, alpha)


def kernel(x, edge_index, W1, b1, W2, b2, alpha):
    src = edge_index[0].astype(jnp.int32)
    dst = edge_index[1].astype(jnp.int32)
    pk = (src | (dst << 16)).reshape(NW, E // NW)
    npd = EPAD - E
    pidx = jnp.arange(npd, dtype=jnp.int32)
    pad = ((pidx * 97) % N | ((N + pidx % (NPAD - N)) << 16)).reshape(NW, npd // NW)
    pkflat = jnp.concatenate([pk, pad], axis=1)
    pkd = pkflat.reshape(NW, NCHD, KD)
    pka = pkflat.reshape(NW, NCHA, KA)
    zrow = jnp.zeros((RPT,), jnp.float32)
    zblk = jnp.zeros((RPT, D), jnp.float32)
    b1r = b1.reshape(1, D)
    b2r = b2.reshape(1, D)
    alr = alpha.reshape(1, D)

    xw1 = _tc_matmul1(x, W1)
    degp = _sc_degree(pkd, zrow)
    degt = degp.T

    y1 = _tc_scale1(xw1, degt)
    agg1 = _sc_agg(y1, pka, zblk)
    y2 = _tc_mid(agg1, degt, b1r, W2)
    agg2 = _sc_agg(y2, pka, zblk)
    return _tc_final(agg2, degt, b2r, alr)


# speedup vs baseline: 1.0186x; 1.0186x over previous
import functools

import jax
import jax.numpy as jnp
from jax import lax
from jax.experimental import pallas as pl
from jax.experimental.pallas import tpu as pltpu
from jax.experimental.pallas import tpu_sc as plsc

N = 10000
E = 320000
D = 128

NC = 2
NS = 16
NW = NC * NS

NPAD = 10240
RPT = NPAD // NS
EPAD = 327680

KD = 128
NCHD = EPAD // NW // KD

KA = 80
EPTA = EPAD // NW
NCHA = EPTA // KA
NPAIR = NCHA // 2

BN = 5000
GRID = N // BN


def _mesh():
    return plsc.VectorSubcoreMesh(
        core_axis_name="c", subcore_axis_name="s",
        num_cores=NC, num_subcores=NS)


def _unpack_chunk(pk, j, sbuf, dbuf, k, soff):
    for g in range(k // 16):
        pv = pk[j, pl.ds(g * 16, 16)]
        sbuf[pl.ds(g * 16, 16)] = (pv & 0xFFFF) + soff
        dbuf[pl.ds(g * 16, 16)] = pv >> 16


def _sc_degree(pkd, zrow):
    @functools.partial(
        pl.kernel,
        out_type=jax.ShapeDtypeStruct((NC, NPAD), jnp.float32),
        mesh=_mesh(),
        scratch_types=[
            pltpu.VMEM((NCHD, KD), jnp.int32),
            pltpu.VMEM((KD,), jnp.int32),
            pltpu.VMEM((KD,), jnp.int32),
            pltpu.VMEM((KD,), jnp.float32),
            pltpu.VMEM_SHARED((NPAD,), jnp.float32),
        ],
    )
    def deg_kernel(pk_h, zrow_h, out_h, pk, sbuf, dbuf, ones_v, deg_sh):
        c = lax.axis_index("c")
        s = lax.axis_index("s")
        w = c * NS + s
        pltpu.sync_copy(zrow_h, deg_sh.at[pl.ds(s * RPT, RPT)])
        pltpu.sync_copy(pk_h.at[w], pk)
        for i in range(KD // 16):
            ones_v[pl.ds(i * 16, 16)] = jnp.ones((16,), jnp.float32)
        plsc.subcore_barrier()

        def body(j, carry):
            _unpack_chunk(pk, j, sbuf, dbuf, KD, 0)
            pltpu.sync_copy(ones_v, deg_sh.at[dbuf], add=True)
            return carry

        lax.fori_loop(0, NCHD, body, 0)
        plsc.subcore_barrier()
        pltpu.sync_copy(deg_sh.at[pl.ds(s * RPT, RPT)],
                        out_h.at[c, pl.ds(s * RPT, RPT)])

    return deg_kernel(pkd, zrow)


def _sc_agg(y, pka, zblk):
    @functools.partial(
        pl.kernel,
        out_type=jax.ShapeDtypeStruct((NC, NPAD, D), jnp.float32),
        mesh=_mesh(),
        scratch_types=[
            pltpu.VMEM((NCHA, KA), jnp.int32),
            pltpu.VMEM((KA,), jnp.int32),
            pltpu.VMEM((KA,), jnp.int32),
            pltpu.VMEM((KA,), jnp.int32),
            pltpu.VMEM((KA,), jnp.int32),
            pltpu.VMEM((KA, D), jnp.float32),
            pltpu.VMEM((KA, D), jnp.float32),
            pltpu.VMEM_SHARED((NPAD, D), jnp.float32),
            pltpu.SemaphoreType.DMA,
            pltpu.SemaphoreType.DMA,
        ],
    )
    def agg_kernel(y_h, pk_h, zblk_h, out_h,
                   pk, sidx0, didx0, sidx1, didx1, rows0, rows1,
                   acc_sh, semg0, semg1):
        c = lax.axis_index("c")
        s = lax.axis_index("s")
        w = c * NS + s
        pltpu.sync_copy(pk_h.at[w], pk)
        _unpack_chunk(pk, 0, sidx0, didx0, KA, 0)
        pltpu.async_copy(y_h.at[sidx0], rows0, semg0)
        _unpack_chunk(pk, 1, sidx1, didx1, KA, 0)
        pltpu.async_copy(y_h.at[sidx1], rows1, semg1)
        last = NS - 1

        @pl.when((c == 0) & (s < last))
        def _():
            pltpu.sync_copy(y_h.at[pl.ds(s * RPT, RPT)],
                            acc_sh.at[pl.ds(s * RPT, RPT)])

        @pl.when((c == 0) & (s == last))
        def _():
            pltpu.sync_copy(y_h.at[pl.ds(last * RPT, N - last * RPT)],
                            acc_sh.at[pl.ds(last * RPT, N - last * RPT)])
            pltpu.sync_copy(zblk_h.at[pl.ds(0, NPAD - N)],
                            acc_sh.at[pl.ds(N, NPAD - N)])

        @pl.when(c == 1)
        def _():
            pltpu.sync_copy(zblk_h, acc_sh.at[pl.ds(s * RPT, RPT)])

        plsc.subcore_barrier()

        def body(i, carry):
            j = 2 * i
            pltpu.make_async_copy(y_h.at[sidx0], rows0, semg0).wait()
            pltpu.sync_copy(rows0, acc_sh.at[didx0], add=True)

            @pl.when(i < NPAIR - 1)
            def _():
                _unpack_chunk(pk, j + 2, sidx0, didx0, KA, 0)
                pltpu.async_copy(y_h.at[sidx0], rows0, semg0)

            pltpu.make_async_copy(y_h.at[sidx1], rows1, semg1).wait()
            pltpu.sync_copy(rows1, acc_sh.at[didx1], add=True)

            @pl.when(i < NPAIR - 1)
            def _():
                _unpack_chunk(pk, j + 3, sidx1, didx1, KA, 0)
                pltpu.async_copy(y_h.at[sidx1], rows1, semg1)

            return carry

        lax.fori_loop(0, NPAIR, body, 0)
        plsc.subcore_barrier()
        pltpu.sync_copy(acc_sh.at[pl.ds(s * RPT, RPT)],
                        out_h.at[c, pl.ds(s * RPT, RPT)])

    return agg_kernel(y, pka, zblk)


def _tc_matmul1(x, W1):
    def body(x_r, w_r, o_r):
        o_r[...] = jnp.dot(x_r[...], w_r[...],
                           preferred_element_type=jnp.float32)

    return pl.pallas_call(
        body,
        grid=(GRID,),
        in_specs=[pl.BlockSpec((BN, D), lambda i: (i, 0)),
                  pl.BlockSpec((D, D), lambda i: (0, 0))],
        out_specs=pl.BlockSpec((BN, D), lambda i: (i, 0)),
        out_shape=jax.ShapeDtypeStruct((N, D), jnp.float32),
    )(x, W1)


def _tc_scale1(xw, degt):
    def body(x_r, dg_r, y_r):
        dinv = lax.rsqrt(dg_r[:, 0:1] + dg_r[:, 1:2] + 1.0)
        y_r[...] = dinv * x_r[...]

    return pl.pallas_call(
        body,
        grid=(GRID,),
        in_specs=[pl.BlockSpec((BN, D), lambda i: (i, 0)),
                  pl.BlockSpec((BN, 2), lambda i: (i, 0))],
        out_specs=pl.BlockSpec((BN, D), lambda i: (i, 0)),
        out_shape=jax.ShapeDtypeStruct((N, D), jnp.float32),
    )(xw, degt)


def _tc_mid(agg, degt, b1, W2):
    def body(a_r, dg_r, b_r, w_r, o_r):
        dinv = lax.rsqrt(dg_r[:, 0:1] + dg_r[:, 1:2] + 1.0)
        z = dinv * (a_r[0] + a_r[1]) + b_r[...]
        o_r[...] = dinv * jnp.dot(z, w_r[...],
                                  preferred_element_type=jnp.float32)

    return pl.pallas_call(
        body,
        grid=(GRID,),
        in_specs=[pl.BlockSpec((NC, BN, D), lambda i: (0, i, 0)),
                  pl.BlockSpec((BN, 2), lambda i: (i, 0)),
                  pl.BlockSpec((1, D), lambda i: (0, 0)),
                  pl.BlockSpec((D, D), lambda i: (0, 0))],
        out_specs=pl.BlockSpec((BN, D), lambda i: (i, 0)),
        out_shape=jax.ShapeDtypeStruct((N, D), jnp.float32),
    )(agg, degt, b1, W2)


def _tc_final(agg, degt, b2, alpha):
    def body(a_r, dg_r, b_r, al_r, o_r):
        dinv = lax.rsqrt(dg_r[:, 0:1] + dg_r[:, 1:2] + 1.0)
        z = dinv * (a_r[0] + a_r[1]) + b_r[...]
        o_r[...] = jnp.where(z >= 0, z, al_r[...] * z)

    return pl.pallas_call(
        body,
        grid=(GRID,),
        in_specs=[pl.BlockSpec((NC, BN, D), lambda i: (0, i, 0)),
                  pl.BlockSpec((BN, 2), lambda i: (i, 0)),
                  pl.BlockSpec((1, D), lambda i: (0, 0)),
                  pl.BlockSpec((1, D), lambda i: (0, 0))],
        out_specs=pl.BlockSpec((BN, D), lambda i: (i, 0)),
        out_shape=jax.ShapeDtypeStruct((N, D), jnp.float32),
    )(agg, degt, b2, alpha)


def kernel(x, edge_index, W1, b1, W2, b2, alpha):
    src = edge_index[0].astype(jnp.int32)
    dst = edge_index[1].astype(jnp.int32)
    pk = (src | (dst << 16)).reshape(NW, E // NW)
    npd = EPAD - E
    pidx = jnp.arange(npd, dtype=jnp.int32)
    pad = ((pidx * 97) % N | ((N + pidx % (NPAD - N)) << 16)).reshape(NW, npd // NW)
    pkflat = jnp.concatenate([pk, pad], axis=1)
    pkd = pkflat.reshape(NW, NCHD, KD)
    pka = pkflat.reshape(NW, NCHA, KA)
    zrow = jnp.zeros((RPT,), jnp.float32)
    zblk = jnp.zeros((RPT, D), jnp.float32)
    b1r = b1.reshape(1, D)
    b2r = b2.reshape(1, D)
    alr = alpha.reshape(1, D)

    xw1 = _tc_matmul1(x, W1)
    degp = _sc_degree(pkd, zrow)
    degt = degp.T

    y1 = _tc_scale1(xw1, degt)
    agg1 = _sc_agg(y1, pka, zblk)
    y2 = _tc_mid(agg1, degt, b1r, W2)
    agg2 = _sc_agg(y2, pka, zblk)
    return _tc_final(agg2, degt, b2r, alr)
